# baseline (speedup 1.0000x reference)
import functools

import jax
import jax.numpy as jnp
from jax import lax
from jax.experimental import pallas as pl
from jax.experimental.pallas import tpu as pltpu
from jax.experimental.pallas import tpu_sc as plsc

_N = 10000
_E = 320000
_G = 64
_NP = 10240
_RB = 512
_NB = _NP // _RB
_NT = 16
_CH = 128
_NCH = 160
_EP = _NT * _NCH * _CH
_DUMMY = _NP - 1
_RPT = _NP // _NT


_SLOTS = 16
_HALF = _SLOTS // 2


def _ring(ustage, acc, idxr, idxc, bufs, semg, sems, nch):
    for b in range(_HALF):
        pltpu.async_copy(ustage.at[idxr.at[b]], bufs[b], semg)

    def outer(i, _):
        for b in range(_SLOTS):
            j = _SLOTS * i + b
            nb = (b + _HALF) % _SLOTS

            @pl.when(j >= _HALF)
            def _():
                pltpu.make_async_copy(
                    bufs[nb], acc.at[idxc.at[j - _HALF]], sems).wait()

            @pl.when(j + _HALF < nch)
            def _():
                pltpu.async_copy(ustage.at[idxr.at[j + _HALF]], bufs[nb], semg)

            pltpu.make_async_copy(ustage.at[idxr.at[j]], bufs[b], semg).wait()
            pltpu.async_copy(bufs[b], acc.at[idxc.at[j]], sems, add=True)
        return 0

    lax.fori_loop(0, nch // _SLOTS, outer, 0)
    for b in range(_HALF):
        pltpu.make_async_copy(
            bufs[(b + _HALF) % _SLOTS],
            acc.at[idxc.at[nch - _HALF + b]], sems).wait()


def _rowloop(n, f):
    def body(i, _):
        f(i)
        return 0
    lax.fori_loop(0, n, body, 0)


_SC_SCRATCH = [
    pltpu.VMEM((_NCH, _CH), jnp.int32),
    pltpu.VMEM((_NCH, _CH), jnp.int32),
    [pltpu.VMEM((_CH, 16), jnp.float32) for _ in range(_SLOTS)],
    pltpu.VMEM((_RPT, 16), jnp.float32),
    pltpu.VMEM((_RPT, 16), jnp.float32),
    pltpu.VMEM((_RPT, 16), jnp.float32),
    pltpu.VMEM((16,), jnp.float32),
    pltpu.VMEM_SHARED((_NP, 16), jnp.float32),
    pltpu.VMEM_SHARED((_NP, 16), jnp.float32),
    pltpu.SemaphoreType.DMA,
    pltpu.SemaphoreType.DMA,
]


def _make_deg_op():
    mesh = plsc.VectorSubcoreMesh(core_axis_name="c", subcore_axis_name="s")

    @functools.partial(
        pl.kernel,
        out_type=jax.ShapeDtypeStruct((_NP,), jnp.float32),
        mesh=mesh,
        scratch_types=[
            pltpu.VMEM((_NCH, _CH), jnp.int32),
            pltpu.VMEM((_CH,), jnp.float32),
            pltpu.VMEM_SHARED((_NP,), jnp.float32),
            pltpu.SemaphoreType.DMA,
        ],
        compiler_params=pltpu.CompilerParams(use_tc_tiling_on_sc=False),
    )
    def deg_op(colp_hbm, ones_hbm, zeros1_hbm, deg_out, idxc, onesv, acc1, sems):
        c = lax.axis_index("c")
        s = lax.axis_index("s")
        rslc = pl.ds(s * _RPT, _RPT)
        pltpu.sync_copy(colp_hbm.at[s], idxc)
        pltpu.sync_copy(ones_hbm, onesv)
        pltpu.sync_copy(zeros1_hbm.at[rslc], acc1.at[rslc])
        plsc.subcore_barrier()

        def body(j, _):
            @pl.when(j >= 4)
            def _():
                pltpu.make_async_copy(onesv, acc1.at[idxc.at[j - 4]], sems).wait()

            pltpu.async_copy(onesv, acc1.at[idxc.at[j]], sems, add=True)
            return 0

        lax.fori_loop(0, _NCH, body, 0)
        for k in range(4):
            pltpu.make_async_copy(
                onesv, acc1.at[idxc.at[_NCH - 4 + k]], sems).wait()
        plsc.subcore_barrier()

        @pl.when(c == 0)
        def _():
            pltpu.sync_copy(acc1.at[rslc], deg_out.at[rslc])

    return deg_op


def _make_horner_op():
    mesh = plsc.VectorSubcoreMesh(core_axis_name="c", subcore_axis_name="s")

    @functools.partial(
        pl.kernel,
        out_type=(jax.ShapeDtypeStruct((_NP, 16), jnp.float32),
                  jax.ShapeDtypeStruct((_NP, 16), jnp.float32)),
        mesh=mesh,
        scratch_types=_SC_SCRATCH,
        compiler_params=pltpu.CompilerParams(use_tc_tiling_on_sc=False),
    )
    def horner_op(dis_hbm, z0h, z1h, z2h, z3h, bs_hbm, rowp_hbm, colp_hbm,
                  zeros_hbm, h_out, s_out,
                  idxr, idxc, bufs, av, bv, disv, bsv, ustage, acc, semg, sems):
        c = lax.axis_index("c")
        s = lax.axis_index("s")
        rslc = pl.ds(s * _RPT, _RPT)
        pltpu.sync_copy(rowp_hbm.at[s], idxr)
        pltpu.sync_copy(colp_hbm.at[s], idxc)
        pltpu.sync_copy(dis_hbm.at[rslc, :], disv)
        pltpu.sync_copy(bs_hbm, bsv)
        pltpu.sync_copy(z3h.at[rslc, :], av)

        def w3(i):
            bv[i, :] = disv[i, :] * av[i, :]

        _rowloop(_RPT, w3)
        pltpu.sync_copy(bv, ustage.at[rslc, :])
        pltpu.sync_copy(zeros_hbm.at[rslc, :], acc.at[rslc, :])
        plsc.subcore_barrier()

        for r, zh in ((2, z2h), (1, z1h), (0, z0h)):
            _ring(ustage, acc, idxr, idxc, bufs, semg, sems, _NCH)
            plsc.subcore_barrier()
            pltpu.sync_copy(acc.at[rslc, :], av)
            pltpu.sync_copy(zh.at[rslc, :], bv)
            if r > 0:
                def comb(i):
                    d = disv[i, :]
                    av[i, :] = d * (bv[i, :] + d * av[i, :])

                _rowloop(_RPT, comb)
                pltpu.sync_copy(av, ustage.at[rslc, :])
                pltpu.sync_copy(zeros_hbm.at[rslc, :], acc.at[rslc, :])
                plsc.subcore_barrier()
            else:
                bias = bsv[:]

                def fin(i):
                    d = disv[i, :]
                    h = jnp.maximum(bv[i, :] + d * av[i, :] + bias, 0.0)
                    av[i, :] = h
                    bv[i, :] = d * h

                _rowloop(_RPT, fin)

                @pl.when(c == 0)
                def _():
                    pltpu.sync_copy(av, h_out.at[rslc, :])
                    pltpu.sync_copy(bv, s_out.at[rslc, :])

    return horner_op


def _make_forward_op():
    mesh = plsc.VectorSubcoreMesh(core_axis_name="c", subcore_axis_name="s")

    @functools.partial(
        pl.kernel,
        out_type=(jax.ShapeDtypeStruct((_NP, 16), jnp.float32),
                  jax.ShapeDtypeStruct((_NP, 16), jnp.float32),
                  jax.ShapeDtypeStruct((_NP, 16), jnp.float32)),
        mesh=mesh,
        scratch_types=_SC_SCRATCH,
        compiler_params=pltpu.CompilerParams(use_tc_tiling_on_sc=False),
    )
    def forward_op(dis_hbm, s0_hbm, rowp_hbm, colp_hbm, zeros_hbm,
                   t1_out, t2_out, t3_out,
                   idxr, idxc, bufs, av, bv, disv, bsv, ustage, acc,
                   semg, sems):
        c = lax.axis_index("c")
        s = lax.axis_index("s")
        rslc = pl.ds(s * _RPT, _RPT)
        pltpu.sync_copy(rowp_hbm.at[s], idxr)
        pltpu.sync_copy(colp_hbm.at[s], idxc)
        pltpu.sync_copy(dis_hbm.at[rslc, :], disv)
        pltpu.sync_copy(s0_hbm.at[rslc, :], bv)
        pltpu.sync_copy(bv, ustage.at[rslc, :])
        pltpu.sync_copy(zeros_hbm.at[rslc, :], acc.at[rslc, :])
        plsc.subcore_barrier()

        for r, t_out in ((1, t1_out), (2, t2_out), (3, t3_out)):
            _ring(ustage, acc, idxr, idxc, bufs, semg, sems, _NCH)
            plsc.subcore_barrier()
            pltpu.sync_copy(acc.at[rslc, :], av)

            def scale_t(i):
                av[i, :] = disv[i, :] * av[i, :]

            _rowloop(_RPT, scale_t)

            @pl.when(c == 0)
            def _():
                pltpu.sync_copy(av, t_out.at[rslc, :])

            if r < 3:
                def scale_w(i):
                    bv[i, :] = disv[i, :] * av[i, :]

                _rowloop(_RPT, scale_w)
                pltpu.sync_copy(bv, ustage.at[rslc, :])
                pltpu.sync_copy(zeros_hbm.at[rslc, :], acc.at[rslc, :])
                plsc.subcore_barrier()

    return forward_op


def _row_spec(w):
    return pl.BlockSpec((_RB, w), lambda i: (i, 0))


def _full_spec(shape):
    return pl.BlockSpec(shape, lambda i: tuple(0 for _ in shape))


def _tca_body(deg_ref, x, w, dis_ref, z0_ref, z1_ref, z2_ref, z3_ref):
    deg = deg_ref[...]
    dis = jnp.where(deg > 0, lax.rsqrt(jnp.maximum(deg, 1e-30)), 0.0)
    dis_ref[...] = jnp.broadcast_to(dis, (_RB, 16))
    z = jnp.dot(x[...], w[...], preferred_element_type=jnp.float32)
    z0_ref[...] = z[:, 0:16]
    z1_ref[...] = z[:, 16:32]
    z2_ref[...] = z[:, 32:48]
    z3_ref[...] = z[:, 48:64]


def _tca(deg2, xp, w1cat):
    return pl.pallas_call(
        _tca_body,
        grid=(_NB,),
        in_specs=[pl.BlockSpec((_RB, 1), lambda i: (i, 0)),
                  _row_spec(128), _full_spec((128, 64))],
        out_specs=[_row_spec(16)] * 5,
        out_shape=[jax.ShapeDtypeStruct((_NP, 16), jnp.float32)] * 5,
    )(deg2, xp, w1cat)


def _tcb_body(h1, t1, t2, t3, w2, w3, b2s, z0_ref, z1_ref, z2_ref, z3_ref):
    tstack = jnp.concatenate([h1[...], t1[...], t2[...], t3[...]], axis=1)
    out2 = jnp.dot(tstack, w2[...], preferred_element_type=jnp.float32) \
        + b2s[...]
    h2 = jnp.maximum(out2, 0.0)
    z3 = jnp.dot(h2, w3[...], preferred_element_type=jnp.float32)
    z0_ref[...] = z3[:, 0:16]
    z1_ref[...] = z3[:, 16:32]
    z2_ref[...] = z3[:, 32:48]
    z3_ref[...] = z3[:, 48:64]


def _tcb(h1, t1, t2, t3, w2cat, w3cat, b2s):
    return pl.pallas_call(
        _tcb_body,
        grid=(_NB,),
        in_specs=[_row_spec(16)] * 4 +
                 [_full_spec((64, 32)), _full_spec((32, 64)), _full_spec((1, 32))],
        out_specs=[_row_spec(16)] * 4,
        out_shape=[jax.ShapeDtypeStruct((_NP, 16), jnp.float32)] * 4,
    )(h1, t1, t2, t3, w2cat, w3cat, b2s)


def _tcc_body(h3, batch, wfc, bfc, out_ref, sums, cnts):
    j = pl.program_id(0)

    @pl.when(j == 0)
    def _():
        sums[...] = jnp.zeros_like(sums)
        cnts[...] = jnp.zeros_like(cnts)

    b = batch[0, 0, :]
    onehot = (b[:, None] == lax.broadcasted_iota(jnp.int32, (_RB, _G), 1)
              ).astype(jnp.float32)
    sums[...] += lax.dot_general(onehot, h3[...], (((0,), (0,)), ((), ())),
                                 preferred_element_type=jnp.float32)
    cnts[...] += jnp.broadcast_to(jnp.sum(onehot, axis=0)[:, None], (_G, 16))

    @pl.when(j == _NB - 1)
    def _():
        pooled = sums[...] / jnp.maximum(cnts[...], 1.0)
        logits = jnp.dot(pooled, wfc[...], preferred_element_type=jnp.float32) \
            + bfc[...]
        m = jnp.max(logits, axis=1, keepdims=True)
        lse = jnp.log(jnp.sum(jnp.exp(logits - m), axis=1, keepdims=True))
        out_ref[...] = (logits - m) - lse


def _tcc(h3, batch3, wfc, bfc):
    return pl.pallas_call(
        _tcc_body,
        grid=(_NB,),
        in_specs=[_row_spec(16),
                  pl.BlockSpec((1, 1, _RB), lambda i: (i, 0, 0)),
                  _full_spec((16, 13)), _full_spec((1, 13))],
        out_specs=pl.BlockSpec((_G, 13), lambda i: (0, 0)),
        out_shape=jax.ShapeDtypeStruct((_G, 13), jnp.float32),
        scratch_shapes=[pltpu.VMEM((_G, 16), jnp.float32),
                        pltpu.VMEM((_G, 16), jnp.float32)],
    )(h3, batch3, wfc, bfc)


def kernel(x, edge_index, batch, W1, b1, W2, b2, W3, b3, Wfc, bfc):
    row = edge_index[0].astype(jnp.int32)
    col = edge_index[1].astype(jnp.int32)
    rowp = jnp.concatenate(
        [row, jnp.zeros((_EP - _E,), jnp.int32)]).reshape(_NT, _NCH, _CH)
    colp = jnp.concatenate(
        [col, jnp.full((_EP - _E,), _DUMMY, jnp.int32)]).reshape(_NT, _NCH, _CH)
    xp = jnp.zeros((_NP, 128), jnp.float32).at[:_N].set(x)
    batch3 = jnp.full((_NP,), _G, jnp.int32).at[:_N].set(
        batch.astype(jnp.int32)).reshape(_NB, 1, _RB)
    ones_ch = jnp.ones((_CH,), jnp.float32)
    zeros1 = jnp.zeros((_NP,), jnp.float32)
    zeros_np = jnp.zeros((_NP, 16), jnp.float32)
    w1cat = jnp.transpose(W1, (1, 0, 2)).reshape(128, 64)
    w2cat = W2.reshape(64, 32)
    w3cat = jnp.transpose(W3, (1, 0, 2)).reshape(32, 64)
    b1s = jnp.sum(b1, axis=0)
    b2s = jnp.sum(b2, axis=0).reshape(1, 32)
    b3s = jnp.sum(b3, axis=0)
    bfc2 = bfc.reshape(1, 13)

    deg = _make_deg_op()(colp, ones_ch, zeros1)
    dis16, z0, z1, z2, z3 = _tca(deg.reshape(_NP, 1), xp, w1cat)

    horner = _make_horner_op()
    h1, s0 = horner(dis16, z0, z1, z2, z3, b1s, rowp, colp, zeros_np)
    t1, t2, t3 = _make_forward_op()(dis16, s0, rowp, colp, zeros_np)
    z0b, z1b, z2b, z3b = _tcb(h1, t1, t2, t3, w2cat, w3cat, b2s)
    h3, _unused = horner(dis16, z0b, z1b, z2b, z3b, b3s, rowp, colp, zeros_np)
    return _tcc(h3, batch3, Wfc, bfc2)


# speedup vs baseline: 1.0612x; 1.0612x over previous
import functools

import jax
import jax.numpy as jnp
from jax import lax
from jax.experimental import pallas as pl
from jax.experimental.pallas import tpu as pltpu
from jax.experimental.pallas import tpu_sc as plsc

_N = 10000
_E = 320000
_G = 64
_NP = 10240
_RB = 512
_NB = _NP // _RB
_NT = 16
_CH = 128
_NCH = 160
_EP = _NT * _NCH * _CH
_DUMMY = _NP - 1
_RPT = _NP // _NT


_SLOTS = 8
_HALF = _SLOTS // 2


def _ring(ustage, acc, idxr, idxc, bufs, semg, sems, nch, start=0):
    for b in range(_HALF):
        pltpu.async_copy(ustage.at[idxr.at[start + b]], bufs[b], semg)

    def outer(i, _):
        for b in range(_SLOTS):
            j = _SLOTS * i + b
            nb = (b + _HALF) % _SLOTS

            @pl.when(j >= _HALF)
            def _():
                pltpu.make_async_copy(
                    bufs[nb], acc.at[idxc.at[start + j - _HALF]], sems).wait()

            @pl.when(j + _HALF < nch)
            def _():
                pltpu.async_copy(ustage.at[idxr.at[start + j + _HALF]], bufs[nb], semg)

            pltpu.make_async_copy(ustage.at[idxr.at[start + j]], bufs[b], semg).wait()
            pltpu.async_copy(bufs[b], acc.at[idxc.at[start + j]], sems, add=True)
        return 0

    lax.fori_loop(0, nch // _SLOTS, outer, 0)
    for b in range(_HALF):
        pltpu.make_async_copy(
            bufs[(b + _HALF) % _SLOTS],
            acc.at[idxc.at[start + nch - _HALF + b]], sems).wait()


def _rowloop(n, f):
    def body(i, _):
        f(i)
        return 0
    lax.fori_loop(0, n, body, 0)


_SC_SCRATCH = [
    pltpu.VMEM((_NCH, _CH), jnp.int32),
    pltpu.VMEM((_NCH, _CH), jnp.int32),
    [pltpu.VMEM((_CH, 16), jnp.float32) for _ in range(_SLOTS)],
    pltpu.VMEM((_RPT, 16), jnp.float32),
    pltpu.VMEM((_RPT, 16), jnp.float32),
    pltpu.VMEM((_RPT, 16), jnp.float32),
    pltpu.VMEM((16,), jnp.float32),
    pltpu.VMEM_SHARED((_NP, 16), jnp.float32),
    pltpu.VMEM_SHARED((_NP, 16), jnp.float32),
    pltpu.SemaphoreType.DMA,
    pltpu.SemaphoreType.DMA,
]


def _make_deg_op():
    mesh = plsc.VectorSubcoreMesh(core_axis_name="c", subcore_axis_name="s")

    @functools.partial(
        pl.kernel,
        out_type=jax.ShapeDtypeStruct((_NP,), jnp.float32),
        mesh=mesh,
        scratch_types=[
            pltpu.VMEM((_NCH, _CH), jnp.int32),
            pltpu.VMEM((_CH,), jnp.float32),
            pltpu.VMEM_SHARED((_NP,), jnp.float32),
            pltpu.SemaphoreType.DMA,
        ],
        compiler_params=pltpu.CompilerParams(use_tc_tiling_on_sc=False),
    )
    def deg_op(colp_hbm, ones_hbm, zeros1_hbm, deg_out, idxc, onesv, acc1, sems):
        c = lax.axis_index("c")
        s = lax.axis_index("s")
        rslc = pl.ds(s * _RPT, _RPT)
        pltpu.sync_copy(colp_hbm.at[s], idxc)
        pltpu.sync_copy(ones_hbm, onesv)
        pltpu.sync_copy(zeros1_hbm.at[rslc], acc1.at[rslc])
        plsc.subcore_barrier()

        def body(j, _):
            @pl.when(j >= 4)
            def _():
                pltpu.make_async_copy(onesv, acc1.at[idxc.at[j - 4]], sems).wait()

            pltpu.async_copy(onesv, acc1.at[idxc.at[j]], sems, add=True)
            return 0

        lax.fori_loop(0, _NCH, body, 0)
        for k in range(4):
            pltpu.make_async_copy(
                onesv, acc1.at[idxc.at[_NCH - 4 + k]], sems).wait()
        plsc.subcore_barrier()

        @pl.when(c == 0)
        def _():
            pltpu.sync_copy(acc1.at[rslc], deg_out.at[rslc])

    return deg_op


def _make_horner_op():
    mesh = plsc.VectorSubcoreMesh(core_axis_name="c", subcore_axis_name="s")

    @functools.partial(
        pl.kernel,
        out_type=jax.ShapeDtypeStruct((2, _NP, 16), jnp.float32),
        mesh=mesh,
        scratch_types=_SC_SCRATCH,
        compiler_params=pltpu.CompilerParams(use_tc_tiling_on_sc=False),
    )
    def horner_op(dis_hbm, z1h, z2h, z3h, rowp_hbm, colp_hbm,
                  zeros_hbm, p_out,
                  idxr, idxc, bufs, av, bv, disv, bsv, ustage, acc, semg, sems):
        c = lax.axis_index("c")
        s = lax.axis_index("s")
        rslc = pl.ds(s * _RPT, _RPT)
        pltpu.sync_copy(rowp_hbm.at[s], idxr)
        pltpu.sync_copy(colp_hbm.at[s], idxc)
        pltpu.sync_copy(dis_hbm.at[rslc, :], disv)
        pltpu.sync_copy(z3h.at[rslc, :], av)

        def w3(i):
            bv[i, :] = disv[i, :] * av[i, :]

        _rowloop(_RPT, w3)
        pltpu.sync_copy(bv, ustage.at[rslc, :])
        pltpu.sync_copy(zeros_hbm.at[rslc, :], acc.at[rslc, :])
        plsc.subcore_barrier()

        for zh in (z2h, z1h):
            _ring(ustage, acc, idxr, idxc, bufs, semg, sems, _NCH)
            plsc.subcore_barrier()
            pltpu.sync_copy(acc.at[rslc, :], av)
            pltpu.sync_copy(zh.at[rslc, :], bv)

            def comb(i):
                d = disv[i, :]
                av[i, :] = d * (bv[i, :] + d * av[i, :])

            _rowloop(_RPT, comb)
            pltpu.sync_copy(av, ustage.at[rslc, :])
            pltpu.sync_copy(zeros_hbm.at[rslc, :], acc.at[rslc, :])
            plsc.subcore_barrier()

        _ring(ustage, acc, idxr, idxc, bufs, semg, sems, _NCH // 2,
              start=c * (_NCH // 2))
        plsc.subcore_barrier()
        pltpu.sync_copy(acc.at[rslc, :], p_out.at[c, rslc, :])

    return horner_op


def _make_forward_op():
    mesh = plsc.VectorSubcoreMesh(core_axis_name="c", subcore_axis_name="s")

    @functools.partial(
        pl.kernel,
        out_type=(jax.ShapeDtypeStruct((_NP, 16), jnp.float32),
                  jax.ShapeDtypeStruct((_NP, 16), jnp.float32),
                  jax.ShapeDtypeStruct((_NP, 16), jnp.float32),
                  jax.ShapeDtypeStruct((2, _NP, 16), jnp.float32)),
        mesh=mesh,
        scratch_types=_SC_SCRATCH,
        compiler_params=pltpu.CompilerParams(use_tc_tiling_on_sc=False),
    )
    def forward_op(dis_hbm, z0h, p_hbm, bs_hbm, rowp_hbm, colp_hbm, zeros_hbm,
                   h_out, t1_out, t2_out, q_out,
                   idxr, idxc, bufs, av, bv, disv, bsv, ustage, acc,
                   semg, sems):
        c = lax.axis_index("c")
        s = lax.axis_index("s")
        rslc = pl.ds(s * _RPT, _RPT)
        pltpu.sync_copy(rowp_hbm.at[s], idxr)
        pltpu.sync_copy(colp_hbm.at[s], idxc)
        pltpu.sync_copy(dis_hbm.at[rslc, :], disv)
        pltpu.sync_copy(bs_hbm, bsv)
        pltpu.sync_copy(p_hbm.at[0, rslc, :], av)
        pltpu.sync_copy(p_hbm.at[1, rslc, :], bv)

        def psum(i):
            bv[i, :] = disv[i, :] * (av[i, :] + bv[i, :])

        _rowloop(_RPT, psum)
        pltpu.sync_copy(z0h.at[rslc, :], av)
        bias = bsv[:]

        def finh(i):
            h = jnp.maximum(av[i, :] + bv[i, :] + bias, 0.0)
            av[i, :] = h
            bv[i, :] = disv[i, :] * h

        _rowloop(_RPT, finh)

        @pl.when(c == 0)
        def _():
            pltpu.sync_copy(av, h_out.at[rslc, :])

        pltpu.sync_copy(bv, ustage.at[rslc, :])
        pltpu.sync_copy(zeros_hbm.at[rslc, :], acc.at[rslc, :])
        plsc.subcore_barrier()

        for t_out in (t1_out, t2_out):
            _ring(ustage, acc, idxr, idxc, bufs, semg, sems, _NCH)
            plsc.subcore_barrier()
            pltpu.sync_copy(acc.at[rslc, :], av)

            def scale_t(i):
                av[i, :] = disv[i, :] * av[i, :]

            _rowloop(_RPT, scale_t)

            @pl.when(c == 0)
            def _():
                pltpu.sync_copy(av, t_out.at[rslc, :])

            def scale_w(i):
                bv[i, :] = disv[i, :] * av[i, :]

            _rowloop(_RPT, scale_w)
            pltpu.sync_copy(bv, ustage.at[rslc, :])
            pltpu.sync_copy(zeros_hbm.at[rslc, :], acc.at[rslc, :])
            plsc.subcore_barrier()

        _ring(ustage, acc, idxr, idxc, bufs, semg, sems, _NCH // 2,
              start=c * (_NCH // 2))
        plsc.subcore_barrier()
        pltpu.sync_copy(acc.at[rslc, :], q_out.at[c, rslc, :])

    return forward_op


def _row_spec(w):
    return pl.BlockSpec((_RB, w), lambda i: (i, 0))


def _full_spec(shape):
    return pl.BlockSpec(shape, lambda i: tuple(0 for _ in shape))


def _tca_body(deg_ref, x, w, dis_ref, z0_ref, z1_ref, z2_ref, z3_ref):
    deg = deg_ref[...]
    dis = jnp.where(deg > 0, lax.rsqrt(jnp.maximum(deg, 1e-30)), 0.0)
    dis_ref[...] = jnp.broadcast_to(dis, (_RB, 16))
    z = jnp.dot(x[...], w[...], preferred_element_type=jnp.float32)
    z0_ref[...] = z[:, 0:16]
    z1_ref[...] = z[:, 16:32]
    z2_ref[...] = z[:, 32:48]
    z3_ref[...] = z[:, 48:64]


def _tca(deg2, xp, w1cat):
    return pl.pallas_call(
        _tca_body,
        grid=(_NB,),
        in_specs=[pl.BlockSpec((_RB, 1), lambda i: (i, 0)),
                  _row_spec(128), _full_spec((128, 64))],
        out_specs=[_row_spec(16)] * 5,
        out_shape=[jax.ShapeDtypeStruct((_NP, 16), jnp.float32)] * 5,
    )(deg2, xp, w1cat)


def _tcb_body(h1, t1, t2, dis, q0, q1, w2, w3, b2s,
              z0_ref, z1_ref, z2_ref, z3_ref):
    t3 = dis[...] * (q0[...] + q1[...])
    tstack = jnp.concatenate([h1[...], t1[...], t2[...], t3], axis=1)
    out2 = jnp.dot(tstack, w2[...], preferred_element_type=jnp.float32) \
        + b2s[...]
    h2 = jnp.maximum(out2, 0.0)
    z3 = jnp.dot(h2, w3[...], preferred_element_type=jnp.float32)
    z0_ref[...] = z3[:, 0:16]
    z1_ref[...] = z3[:, 16:32]
    z2_ref[...] = z3[:, 32:48]
    z3_ref[...] = z3[:, 48:64]


def _tcb(h1, t1, t2, dis16, q0, q1, w2cat, w3cat, b2s):
    return pl.pallas_call(
        _tcb_body,
        grid=(_NB,),
        in_specs=[_row_spec(16)] * 6 +
                 [_full_spec((64, 32)), _full_spec((32, 64)), _full_spec((1, 32))],
        out_specs=[_row_spec(16)] * 4,
        out_shape=[jax.ShapeDtypeStruct((_NP, 16), jnp.float32)] * 4,
    )(h1, t1, t2, dis16, q0, q1, w2cat, w3cat, b2s)


def _tcc_body(z0, dis, p0, p1, b3s, batch, wfc, bfc, out_ref, sums, cnts):
    j = pl.program_id(0)

    @pl.when(j == 0)
    def _():
        sums[...] = jnp.zeros_like(sums)
        cnts[...] = jnp.zeros_like(cnts)

    h3 = jnp.maximum(
        z0[...] + dis[...] * (p0[...] + p1[...]) + b3s[...], 0.0)
    b = batch[0, 0, :]
    onehot = (b[:, None] == lax.broadcasted_iota(jnp.int32, (_RB, _G), 1)
              ).astype(jnp.float32)
    sums[...] += lax.dot_general(onehot, h3, (((0,), (0,)), ((), ())),
                                 preferred_element_type=jnp.float32)
    cnts[...] += jnp.broadcast_to(jnp.sum(onehot, axis=0)[:, None], (_G, 16))

    @pl.when(j == _NB - 1)
    def _():
        pooled = sums[...] / jnp.maximum(cnts[...], 1.0)
        logits = jnp.dot(pooled, wfc[...], preferred_element_type=jnp.float32) \
            + bfc[...]
        m = jnp.max(logits, axis=1, keepdims=True)
        lse = jnp.log(jnp.sum(jnp.exp(logits - m), axis=1, keepdims=True))
        out_ref[...] = (logits - m) - lse


def _tcc(z0b, dis16, p0, p1, b3s, batch3, wfc, bfc):
    return pl.pallas_call(
        _tcc_body,
        grid=(_NB,),
        in_specs=[_row_spec(16)] * 4 +
                 [_full_spec((1, 16)),
                  pl.BlockSpec((1, 1, _RB), lambda i: (i, 0, 0)),
                  _full_spec((16, 13)), _full_spec((1, 13))],
        out_specs=pl.BlockSpec((_G, 13), lambda i: (0, 0)),
        out_shape=jax.ShapeDtypeStruct((_G, 13), jnp.float32),
        scratch_shapes=[pltpu.VMEM((_G, 16), jnp.float32),
                        pltpu.VMEM((_G, 16), jnp.float32)],
    )(z0b, dis16, p0, p1, b3s, batch3, wfc, bfc)


def kernel(x, edge_index, batch, W1, b1, W2, b2, W3, b3, Wfc, bfc):
    row = edge_index[0].astype(jnp.int32)
    col = edge_index[1].astype(jnp.int32)
    rowp = jnp.concatenate(
        [row, jnp.zeros((_EP - _E,), jnp.int32)]).reshape(_NT, _NCH, _CH)
    colp = jnp.concatenate(
        [col, jnp.full((_EP - _E,), _DUMMY, jnp.int32)]).reshape(_NT, _NCH, _CH)
    xp = jnp.zeros((_NP, 128), jnp.float32).at[:_N].set(x)
    batch3 = jnp.full((_NP,), _G, jnp.int32).at[:_N].set(
        batch.astype(jnp.int32)).reshape(_NB, 1, _RB)
    ones_ch = jnp.ones((_CH,), jnp.float32)
    zeros1 = jnp.zeros((_NP,), jnp.float32)
    zeros_np = jnp.zeros((_NP, 16), jnp.float32)
    w1cat = jnp.transpose(W1, (1, 0, 2)).reshape(128, 64)
    w2cat = W2.reshape(64, 32)
    w3cat = jnp.transpose(W3, (1, 0, 2)).reshape(32, 64)
    b1s = jnp.sum(b1, axis=0)
    b2s = jnp.sum(b2, axis=0).reshape(1, 32)
    b3s = jnp.sum(b3, axis=0)
    bfc2 = bfc.reshape(1, 13)

    deg = _make_deg_op()(colp, ones_ch, zeros1)
    dis16, z0, z1, z2, z3 = _tca(deg.reshape(_NP, 1), xp, w1cat)

    horner = _make_horner_op()
    p = horner(dis16, z1, z2, z3, rowp, colp, zeros_np)
    h1, t1, t2, q = _make_forward_op()(dis16, z0, p, b1s, rowp, colp, zeros_np)
    z0b, z1b, z2b, z3b = _tcb(h1, t1, t2, dis16, q[0], q[1],
                              w2cat, w3cat, b2s)
    pb = horner(dis16, z1b, z2b, z3b, rowp, colp, zeros_np)
    return _tcc(z0b, dis16, pb[0], pb[1], b3s.reshape(1, 16), batch3,
                Wfc, bfc2)


# speedup vs baseline: 1.0873x; 1.0245x over previous
import functools

import jax
import jax.numpy as jnp
from jax import lax
from jax.experimental import pallas as pl
from jax.experimental.pallas import tpu as pltpu
from jax.experimental.pallas import tpu_sc as plsc

_N = 10000
_E = 320000
_G = 64
_NP = 10240
_RB = 512
_NB = _NP // _RB
_NT = 16
_CH = 128
_NCH = 160
_EP = _NT * _NCH * _CH
_DUMMY = _NP - 1
_RPT = _NP // _NT


_SLOTS = 8
_HALF = _SLOTS // 2


def _ring(ustage, acc, idxr, idxc, bufs, semg, sems, nch, start=0):
    for b in range(_HALF):
        pltpu.async_copy(ustage.at[idxr.at[start + b]], bufs[b], semg)

    def outer(i, _):
        for b in range(_SLOTS):
            j = _SLOTS * i + b
            nb = (b + _HALF) % _SLOTS

            @pl.when(j >= _HALF)
            def _():
                pltpu.make_async_copy(
                    bufs[nb], acc.at[idxc.at[start + j - _HALF]], sems).wait()

            @pl.when(j + _HALF < nch)
            def _():
                pltpu.async_copy(ustage.at[idxr.at[start + j + _HALF]], bufs[nb], semg)

            pltpu.make_async_copy(ustage.at[idxr.at[start + j]], bufs[b], semg).wait()
            pltpu.async_copy(bufs[b], acc.at[idxc.at[start + j]], sems, add=True)
        return 0

    lax.fori_loop(0, nch // _SLOTS, outer, 0)
    for b in range(_HALF):
        pltpu.make_async_copy(
            bufs[(b + _HALF) % _SLOTS],
            acc.at[idxc.at[start + nch - _HALF + b]], sems).wait()


def _rowloop(n, f):
    def body(i, _):
        f(i)
        return 0
    lax.fori_loop(0, n, body, 0)


_SC_SCRATCH = [
    pltpu.VMEM((_NCH, _CH), jnp.int32),
    pltpu.VMEM((_NCH, _CH), jnp.int32),
    [pltpu.VMEM((_CH, 16), jnp.float32) for _ in range(_SLOTS)],
    pltpu.VMEM((_RPT, 16), jnp.float32),
    pltpu.VMEM((_RPT, 16), jnp.float32),
    pltpu.VMEM((_RPT, 16), jnp.float32),
    pltpu.VMEM((16,), jnp.float32),
    pltpu.VMEM_SHARED((_NP, 16), jnp.float32),
    pltpu.VMEM_SHARED((_NP, 16), jnp.float32),
    pltpu.SemaphoreType.DMA,
    pltpu.SemaphoreType.DMA,
]


def _make_horner_op(first):
    mesh = plsc.VectorSubcoreMesh(core_axis_name="c", subcore_axis_name="s")

    if first:
        outs = (jax.ShapeDtypeStruct((2, _NP, 16), jnp.float32),
                jax.ShapeDtypeStruct((_NP, 16), jnp.float32))
        extra_scratch = [
            pltpu.VMEM((_CH,), jnp.float32),
            pltpu.VMEM((_RPT,), jnp.float32),
            pltpu.VMEM_SHARED((_NP,), jnp.float32),
        ]
    else:
        outs = jax.ShapeDtypeStruct((2, _NP, 16), jnp.float32)
        extra_scratch = []

    def tail(c, s, rslc, z1h, z2h, z3h, zeros_hbm, p_out,
             idxr, idxc, bufs, av, bv, disv, ustage, acc, semg, sems):
        pltpu.sync_copy(z3h.at[rslc, :], av)

        def w3(i):
            bv[i, :] = disv[i, :] * av[i, :]

        _rowloop(_RPT, w3)
        pltpu.sync_copy(bv, ustage.at[rslc, :])
        pltpu.sync_copy(zeros_hbm.at[rslc, :], acc.at[rslc, :])
        plsc.subcore_barrier()

        for zh in (z2h, z1h):
            _ring(ustage, acc, idxr, idxc, bufs, semg, sems, _NCH)
            plsc.subcore_barrier()
            pltpu.sync_copy(acc.at[rslc, :], av)
            pltpu.sync_copy(zh.at[rslc, :], bv)

            def comb(i):
                d = disv[i, :]
                av[i, :] = d * (bv[i, :] + d * av[i, :])

            _rowloop(_RPT, comb)
            pltpu.sync_copy(av, ustage.at[rslc, :])
            pltpu.sync_copy(zeros_hbm.at[rslc, :], acc.at[rslc, :])
            plsc.subcore_barrier()

        _ring(ustage, acc, idxr, idxc, bufs, semg, sems, _NCH // 2,
              start=c * (_NCH // 2))
        plsc.subcore_barrier()
        pltpu.sync_copy(acc.at[rslc, :], p_out.at[c, rslc, :])

    if first:
        @functools.partial(
            pl.kernel,
            out_type=outs,
            mesh=mesh,
            scratch_types=_SC_SCRATCH + extra_scratch,
            compiler_params=pltpu.CompilerParams(use_tc_tiling_on_sc=False),
        )
        def horner_first(z1h, z2h, z3h, rowp_hbm, colp_hbm, zeros_hbm,
                         ones_hbm, zeros1_hbm, p_out, dis_out,
                         idxr, idxc, bufs, av, bv, disv, bsv, ustage, acc,
                         semg, sems, onesv, degv, acc1):
            c = lax.axis_index("c")
            s = lax.axis_index("s")
            rslc = pl.ds(s * _RPT, _RPT)
            r1 = pl.ds(s * _RPT, _RPT)
            pltpu.sync_copy(rowp_hbm.at[s], idxr)
            pltpu.sync_copy(colp_hbm.at[s], idxc)
            pltpu.sync_copy(ones_hbm, onesv)
            pltpu.sync_copy(zeros1_hbm.at[r1], acc1.at[r1])
            plsc.subcore_barrier()

            def dbody(j, _):
                @pl.when(j >= 4)
                def _():
                    pltpu.make_async_copy(
                        onesv, acc1.at[idxc.at[j - 4]], sems).wait()

                pltpu.async_copy(onesv, acc1.at[idxc.at[j]], sems, add=True)
                return 0

            lax.fori_loop(0, _NCH, dbody, 0)
            for k in range(4):
                pltpu.make_async_copy(
                    onesv, acc1.at[idxc.at[_NCH - 4 + k]], sems).wait()
            plsc.subcore_barrier()
            pltpu.sync_copy(acc1.at[r1], degv)

            def newton(g):
                d = degv[pl.ds(g * 16, 16)]
                dc = jnp.maximum(d, 1e-30)
                seed = 0x5F3759DF - (
                    lax.bitcast_convert_type(dc, jnp.int32) >> 1)
                y = lax.bitcast_convert_type(seed, jnp.float32)
                y = y * (1.5 - 0.5 * dc * y * y)
                y = y * (1.5 - 0.5 * dc * y * y)
                y = y * (1.5 - 0.5 * dc * y * y)
                y = jnp.where(d > 0.0, y, 0.0)
                for lane in range(16):
                    disv[g * 16 + lane, :] = jnp.broadcast_to(y[lane], (16,))

            _rowloop(_RPT // 16, newton)

            @pl.when(c == 0)
            def _():
                pltpu.sync_copy(disv, dis_out.at[rslc, :])

            tail(c, s, rslc, z1h, z2h, z3h, zeros_hbm, p_out,
                 idxr, idxc, bufs, av, bv, disv, ustage, acc, semg, sems)

        return horner_first

    @functools.partial(
        pl.kernel,
        out_type=outs,
        mesh=mesh,
        scratch_types=_SC_SCRATCH,
        compiler_params=pltpu.CompilerParams(use_tc_tiling_on_sc=False),
    )
    def horner_op(dis_hbm, z1h, z2h, z3h, rowp_hbm, colp_hbm, zeros_hbm,
                  p_out,
                  idxr, idxc, bufs, av, bv, disv, bsv, ustage, acc,
                  semg, sems):
        c = lax.axis_index("c")
        s = lax.axis_index("s")
        rslc = pl.ds(s * _RPT, _RPT)
        pltpu.sync_copy(rowp_hbm.at[s], idxr)
        pltpu.sync_copy(colp_hbm.at[s], idxc)
        pltpu.sync_copy(dis_hbm.at[rslc, :], disv)
        tail(c, s, rslc, z1h, z2h, z3h, zeros_hbm, p_out,
             idxr, idxc, bufs, av, bv, disv, ustage, acc, semg, sems)

    return horner_op


def _make_forward_op():
    mesh = plsc.VectorSubcoreMesh(core_axis_name="c", subcore_axis_name="s")

    @functools.partial(
        pl.kernel,
        out_type=(jax.ShapeDtypeStruct((_NP, 16), jnp.float32),
                  jax.ShapeDtypeStruct((_NP, 16), jnp.float32),
                  jax.ShapeDtypeStruct((_NP, 16), jnp.float32),
                  jax.ShapeDtypeStruct((2, _NP, 16), jnp.float32)),
        mesh=mesh,
        scratch_types=_SC_SCRATCH,
        compiler_params=pltpu.CompilerParams(use_tc_tiling_on_sc=False),
    )
    def forward_op(dis_hbm, z0h, p_hbm, bs_hbm, rowp_hbm, colp_hbm, zeros_hbm,
                   h_out, t1_out, t2_out, q_out,
                   idxr, idxc, bufs, av, bv, disv, bsv, ustage, acc,
                   semg, sems):
        c = lax.axis_index("c")
        s = lax.axis_index("s")
        rslc = pl.ds(s * _RPT, _RPT)
        pltpu.sync_copy(rowp_hbm.at[s], idxr)
        pltpu.sync_copy(colp_hbm.at[s], idxc)
        pltpu.sync_copy(dis_hbm.at[rslc, :], disv)
        pltpu.sync_copy(bs_hbm, bsv)
        pltpu.sync_copy(p_hbm.at[0, rslc, :], av)
        pltpu.sync_copy(p_hbm.at[1, rslc, :], bv)

        def psum(i):
            bv[i, :] = disv[i, :] * (av[i, :] + bv[i, :])

        _rowloop(_RPT, psum)
        pltpu.sync_copy(z0h.at[rslc, :], av)
        bias = bsv[:]

        def finh(i):
            h = jnp.maximum(av[i, :] + bv[i, :] + bias, 0.0)
            av[i, :] = h
            bv[i, :] = disv[i, :] * h

        _rowloop(_RPT, finh)

        @pl.when(c == 0)
        def _():
            pltpu.sync_copy(av, h_out.at[rslc, :])

        pltpu.sync_copy(bv, ustage.at[rslc, :])
        pltpu.sync_copy(zeros_hbm.at[rslc, :], acc.at[rslc, :])
        plsc.subcore_barrier()

        for t_out in (t1_out, t2_out):
            _ring(ustage, acc, idxr, idxc, bufs, semg, sems, _NCH)
            plsc.subcore_barrier()
            pltpu.sync_copy(acc.at[rslc, :], av)

            def scale_t(i):
                av[i, :] = disv[i, :] * av[i, :]

            _rowloop(_RPT, scale_t)

            @pl.when(c == 0)
            def _():
                pltpu.sync_copy(av, t_out.at[rslc, :])

            def scale_w(i):
                bv[i, :] = disv[i, :] * av[i, :]

            _rowloop(_RPT, scale_w)
            pltpu.sync_copy(bv, ustage.at[rslc, :])
            pltpu.sync_copy(zeros_hbm.at[rslc, :], acc.at[rslc, :])
            plsc.subcore_barrier()

        _ring(ustage, acc, idxr, idxc, bufs, semg, sems, _NCH // 2,
              start=c * (_NCH // 2))
        plsc.subcore_barrier()
        pltpu.sync_copy(acc.at[rslc, :], q_out.at[c, rslc, :])

    return forward_op


def _row_spec(w):
    return pl.BlockSpec((_RB, w), lambda i: (i, 0))


def _full_spec(shape):
    return pl.BlockSpec(shape, lambda i: tuple(0 for _ in shape))


def _tca_body(x, w, z0_ref, z1_ref, z2_ref, z3_ref):
    z = jnp.dot(x[...], w[...], preferred_element_type=jnp.float32)
    z0_ref[...] = z[:, 0:16]
    z1_ref[...] = z[:, 16:32]
    z2_ref[...] = z[:, 32:48]
    z3_ref[...] = z[:, 48:64]


def _tca(xp, w1cat):
    return pl.pallas_call(
        _tca_body,
        grid=(_NB,),
        in_specs=[_row_spec(128), _full_spec((128, 64))],
        out_specs=[_row_spec(16)] * 4,
        out_shape=[jax.ShapeDtypeStruct((_NP, 16), jnp.float32)] * 4,
    )(xp, w1cat)


def _tcb_body(h1, t1, t2, dis, q0, q1, w2, w3, b2s,
              z0_ref, z1_ref, z2_ref, z3_ref):
    t3 = dis[...] * (q0[...] + q1[...])
    tstack = jnp.concatenate([h1[...], t1[...], t2[...], t3], axis=1)
    out2 = jnp.dot(tstack, w2[...], preferred_element_type=jnp.float32) \
        + b2s[...]
    h2 = jnp.maximum(out2, 0.0)
    z3 = jnp.dot(h2, w3[...], preferred_element_type=jnp.float32)
    z0_ref[...] = z3[:, 0:16]
    z1_ref[...] = z3[:, 16:32]
    z2_ref[...] = z3[:, 32:48]
    z3_ref[...] = z3[:, 48:64]


def _tcb(h1, t1, t2, dis16, q0, q1, w2cat, w3cat, b2s):
    return pl.pallas_call(
        _tcb_body,
        grid=(_NB,),
        in_specs=[_row_spec(16)] * 6 +
                 [_full_spec((64, 32)), _full_spec((32, 64)), _full_spec((1, 32))],
        out_specs=[_row_spec(16)] * 4,
        out_shape=[jax.ShapeDtypeStruct((_NP, 16), jnp.float32)] * 4,
    )(h1, t1, t2, dis16, q0, q1, w2cat, w3cat, b2s)


def _tcc_body(z0, dis, p0, p1, b3s, batch, wfc, bfc, out_ref, sums, cnts):
    j = pl.program_id(0)

    @pl.when(j == 0)
    def _():
        sums[...] = jnp.zeros_like(sums)
        cnts[...] = jnp.zeros_like(cnts)

    h3 = jnp.maximum(
        z0[...] + dis[...] * (p0[...] + p1[...]) + b3s[...], 0.0)
    b = batch[0, 0, :]
    onehot = (b[:, None] == lax.broadcasted_iota(jnp.int32, (_RB, _G), 1)
              ).astype(jnp.float32)
    sums[...] += lax.dot_general(onehot, h3, (((0,), (0,)), ((), ())),
                                 preferred_element_type=jnp.float32)
    cnts[...] += jnp.broadcast_to(jnp.sum(onehot, axis=0)[:, None], (_G, 16))

    @pl.when(j == _NB - 1)
    def _():
        pooled = sums[...] / jnp.maximum(cnts[...], 1.0)
        logits = jnp.dot(pooled, wfc[...], preferred_element_type=jnp.float32) \
            + bfc[...]
        m = jnp.max(logits, axis=1, keepdims=True)
        lse = jnp.log(jnp.sum(jnp.exp(logits - m), axis=1, keepdims=True))
        out_ref[...] = (logits - m) - lse


def _tcc(z0b, dis16, p0, p1, b3s, batch3, wfc, bfc):
    return pl.pallas_call(
        _tcc_body,
        grid=(_NB,),
        in_specs=[_row_spec(16)] * 4 +
                 [_full_spec((1, 16)),
                  pl.BlockSpec((1, 1, _RB), lambda i: (i, 0, 0)),
                  _full_spec((16, 13)), _full_spec((1, 13))],
        out_specs=pl.BlockSpec((_G, 13), lambda i: (0, 0)),
        out_shape=jax.ShapeDtypeStruct((_G, 13), jnp.float32),
        scratch_shapes=[pltpu.VMEM((_G, 16), jnp.float32),
                        pltpu.VMEM((_G, 16), jnp.float32)],
    )(z0b, dis16, p0, p1, b3s, batch3, wfc, bfc)


def kernel(x, edge_index, batch, W1, b1, W2, b2, W3, b3, Wfc, bfc):
    row = edge_index[0].astype(jnp.int32)
    col = edge_index[1].astype(jnp.int32)
    rowp = jnp.concatenate(
        [row, jnp.zeros((_EP - _E,), jnp.int32)]).reshape(_NT, _NCH, _CH)
    colp = jnp.concatenate(
        [col, jnp.full((_EP - _E,), _DUMMY, jnp.int32)]).reshape(_NT, _NCH, _CH)
    xp = jnp.zeros((_NP, 128), jnp.float32).at[:_N].set(x)
    batch3 = jnp.full((_NP,), _G, jnp.int32).at[:_N].set(
        batch.astype(jnp.int32)).reshape(_NB, 1, _RB)
    ones_ch = jnp.ones((_CH,), jnp.float32)
    zeros1 = jnp.zeros((_NP,), jnp.float32)
    zeros_np = jnp.zeros((_NP, 16), jnp.float32)
    w1cat = jnp.transpose(W1, (1, 0, 2)).reshape(128, 64)
    w2cat = W2.reshape(64, 32)
    w3cat = jnp.transpose(W3, (1, 0, 2)).reshape(32, 64)
    b1s = jnp.sum(b1, axis=0)
    b2s = jnp.sum(b2, axis=0).reshape(1, 32)
    b3s = jnp.sum(b3, axis=0)
    bfc2 = bfc.reshape(1, 13)

    z0, z1, z2, z3 = _tca(xp, w1cat)
    p, dis16 = _make_horner_op(True)(z1, z2, z3, rowp, colp, zeros_np,
                                     ones_ch, zeros1)
    h1, t1, t2, q = _make_forward_op()(dis16, z0, p, b1s, rowp, colp, zeros_np)
    z0b, z1b, z2b, z3b = _tcb(h1, t1, t2, dis16, q[0], q[1],
                              w2cat, w3cat, b2s)
    pb = _make_horner_op(False)(dis16, z1b, z2b, z3b, rowp, colp, zeros_np)
    return _tcc(z0b, dis16, pb[0], pb[1], b3s.reshape(1, 16), batch3,
                Wfc, bfc2)
